# baseline (speedup 1.0000x reference)
import dataclasses
import functools

import jax
import jax.numpy as jnp
from jax import lax
from jax.experimental import pallas as pl
from jax.experimental.pallas import tpu as pltpu
from jax.experimental.pallas import tpu_sc as plsc

_NC = 2
_NS = 16
_NW = _NC * _NS
_G = 128
_LANES = 16
_NBUF = 4
_QS = 129


def _compiler_params():
    cp = pltpu.CompilerParams(use_tc_tiling_on_sc=False)
    if "needs_layout_passes" in pltpu.CompilerParams.__dataclass_fields__:
        cp = dataclasses.replace(cp, needs_layout_passes=False)
    return cp


@functools.partial(jax.jit, static_argnames=("seq", "batch", "emb"))
def _sc_embed(table, idx2, off2, pe2, *, seq, batch, emb):
    n_flat = seq * batch
    nchunks = n_flat // _G
    per_w = nchunks // _NW
    cpl = batch // _G
    dt_n = emb // 8
    dc_n = emb // _LANES
    emb_pad = table.shape[1]

    mesh = plsc.VectorSubcoreMesh(core_axis_name="c", subcore_axis_name="s")

    @functools.partial(
        pl.kernel,
        out_type=jax.ShapeDtypeStruct((seq, dt_n, cpl, 8, _G), jnp.float32),
        mesh=mesh,
        scratch_types=[
            pltpu.VMEM((seq, emb), jnp.float32),
            pltpu.VMEM((per_w, _G), jnp.int32),
            pltpu.VMEM((_NBUF, _G), jnp.int32),
            pltpu.VMEM((_NBUF, _G, emb_pad), jnp.float32),
            pltpu.VMEM((2, emb // 8, 8, _QS), jnp.float32),
            pltpu.SemaphoreType.DMA,
            pltpu.SemaphoreType.DMA,
            pltpu.SemaphoreType.DMA,
            pltpu.SemaphoreType.DMA,
            pltpu.SemaphoreType.DMA,
            pltpu.SemaphoreType.DMA,
            pltpu.SemaphoreType.DMA,
            pltpu.SemaphoreType.DMA,
            pltpu.SemaphoreType.DMA,
            pltpu.SemaphoreType.DMA,
        ],
        compiler_params=_compiler_params(),
    )
    def k(table_hbm, idx_hbm, off_hbm, pe_hbm, out_hbm, pe_v, idx_v, off_v,
          rows_v, q_v, sg0, sg1, sg2, sg3, so0, so1, sf0, sf1, sf2, sf3):
        sg = (sg0, sg1, sg2, sg3)
        so = (so0, so1)
        sf = (sf0, sf1, sf2, sf3)
        wid = lax.axis_index("s") * _NC + lax.axis_index("c")
        base = wid * per_w
        pltpu.sync_copy(pe_hbm, pe_v)
        pltpu.sync_copy(idx_hbm.at[pl.ds(base, per_w)], idx_v)

        def gather_copy(g, kb):
            return pltpu.make_async_copy(
                table_hbm.at[idx_v.at[g]], rows_v.at[kb], sg[kb])

        def off_copy(g, kb):
            return pltpu.make_async_copy(
                off_hbm.at[base + g], off_v.at[kb], sf[kb])

        def out_copy(g, ob):
            gi = base + g
            l = gi // cpl
            bt = lax.rem(gi, cpl)
            return pltpu.make_async_copy(
                q_v.at[ob, :, :, pl.ds(0, _G)],
                out_hbm.at[l, :, bt], so[ob])

        iota16 = lax.iota(jnp.int32, 16)
        d_rows = [iota16 + (dc * _LANES) for dc in range(dc_n)]
        d_hi = [r // 8 for r in d_rows]
        d_lo = [lax.rem(r, 8) for r in d_rows]
        zeros16 = jnp.zeros((16,), jnp.int32)

        def compute(g, kb, ob):
            gi = base + g
            kb_splat = zeros16 + kb
            l_splat = zeros16 + gi // cpl
            rows = rows_v.at[kb]
            q = q_v.at[ob]
            pe_regs = [
                plsc.load_gather(pe_v, [l_splat, d_rows[dc]])
                for dc in range(dc_n)
            ]

            @plsc.parallel_loop(0, _G, unroll=8)
            def _(b):
                b_splat = zeros16 + b
                h_splat = plsc.load_gather(off_v, [kb_splat, b_splat])
                for dc in range(dc_n):
                    vals = plsc.load_gather(
                        rows, [b_splat, d_rows[dc] + h_splat])
                    plsc.store_scatter(
                        q, [d_hi[dc], d_lo[dc], b_splat],
                        vals + pe_regs[dc])

        for j in range(_NBUF - 1):
            gather_copy(j, j).start()
            off_copy(j, j).start()

        @pl.loop(0, per_w, step=_NBUF)
        def _(g0):
            for kk in range(_NBUF):
                g = g0 + kk
                gather_copy(g, kk).wait()
                off_copy(g, kk).wait()

                @pl.when(g + (_NBUF - 1) < per_w)
                def _():
                    gather_copy(g + (_NBUF - 1), (kk + (_NBUF - 1)) % _NBUF).start()
                    off_copy(g + (_NBUF - 1), (kk + (_NBUF - 1)) % _NBUF).start()

                @pl.when(g >= 2)
                def _():
                    out_copy(g - 2, kk % 2).wait()

                compute(g, kb=kk, ob=kk % 2)
                out_copy(g, kk % 2).start()

        out_copy(per_w - 2, (per_w - 2) % 2).wait()
        out_copy(per_w - 1, (per_w - 1) % 2).wait()

    return k(table, idx2, off2, pe2)


_TBLK = 4096


def _tc_pack_body(t_ref, out_ref):
    x = t_ref[...]
    half = x.shape[0] // 2
    out_ref[:, : x.shape[1]] = x[:half]
    out_ref[:, x.shape[1]:] = x[half:]


def _tc_pack(table):
    vocab, emb = table.shape
    nblk = pl.cdiv(vocab, _TBLK)
    return pl.pallas_call(
        _tc_pack_body,
        grid=(nblk,),
        in_specs=[pl.BlockSpec((_TBLK, emb), lambda i: (i, 0))],
        out_specs=pl.BlockSpec((_TBLK // 2, 2 * emb), lambda i: (i, 0)),
        out_shape=jax.ShapeDtypeStruct((nblk * (_TBLK // 2), 2 * emb),
                                       jnp.float32),
    )(table)


def kernel(input, table, pe):
    batch, seq = input.shape
    emb = table.shape[1]
    idx2 = input.T.reshape(seq * batch // _G, _G)
    pe2 = pe.reshape(pe.shape[0], emb)[:seq]
    half = _TBLK // 2
    j = idx2 % _TBLK
    off2 = (j // half) * emb
    idx2 = (idx2 // _TBLK) * half + (j % half)
    tpack = _tc_pack(table)
    q = _sc_embed(tpack, idx2, off2, pe2, seq=seq, batch=batch, emb=emb)
    return q.transpose(0, 2, 4, 1, 3).reshape(seq, batch, emb)


# speedup vs baseline: 1.7013x; 1.7013x over previous
import dataclasses
import functools

import jax
import jax.numpy as jnp
from jax import lax
from jax.experimental import pallas as pl
from jax.experimental.pallas import tpu as pltpu
from jax.experimental.pallas import tpu_sc as plsc

_NC = 2
_NS = 16
_NW = _NC * _NS
_G = 128
_LANES = 16
_NBUF = 4
_QS = 129


def _compiler_params():
    cp = pltpu.CompilerParams(use_tc_tiling_on_sc=False)
    if "needs_layout_passes" in pltpu.CompilerParams.__dataclass_fields__:
        cp = dataclasses.replace(cp, needs_layout_passes=False)
    return cp


@functools.partial(jax.jit, static_argnames=("seq", "batch", "emb"))
def _sc_embed(table, idx2, off2, pe2, *, seq, batch, emb):
    n_flat = seq * batch
    nchunks = n_flat // _G
    per_w = nchunks // _NW
    cpl = batch // _G
    dt_n = emb // 8
    dc_n = emb // _LANES
    emb_pad = table.shape[1]

    mesh = plsc.VectorSubcoreMesh(core_axis_name="c", subcore_axis_name="s")

    @functools.partial(
        pl.kernel,
        out_type=jax.ShapeDtypeStruct((seq, dt_n, cpl, 8, _G), jnp.float32),
        mesh=mesh,
        scratch_types=[
            pltpu.VMEM((seq, emb), jnp.float32),
            pltpu.VMEM((per_w, _G), jnp.int32),
            pltpu.VMEM((_NBUF, _G), jnp.int32),
            pltpu.VMEM((_NBUF, _G, emb_pad), jnp.float32),
            pltpu.VMEM((2, emb // 8, 8, _QS), jnp.float32),
            pltpu.SemaphoreType.DMA,
            pltpu.SemaphoreType.DMA,
            pltpu.SemaphoreType.DMA,
            pltpu.SemaphoreType.DMA,
            pltpu.SemaphoreType.DMA,
            pltpu.SemaphoreType.DMA,
            pltpu.SemaphoreType.DMA,
            pltpu.SemaphoreType.DMA,
            pltpu.SemaphoreType.DMA,
            pltpu.SemaphoreType.DMA,
        ],
        compiler_params=_compiler_params(),
    )
    def k(table_hbm, idx_hbm, off_hbm, pe_hbm, out_hbm, pe_v, idx_v, off_v,
          rows_v, q_v, sg0, sg1, sg2, sg3, so0, so1, sf0, sf1, sf2, sf3):
        sg = (sg0, sg1, sg2, sg3)
        so = (so0, so1)
        sf = (sf0, sf1, sf2, sf3)
        wid = lax.axis_index("s") * _NC + lax.axis_index("c")
        base = wid * per_w
        pltpu.sync_copy(pe_hbm, pe_v)
        pltpu.sync_copy(idx_hbm.at[pl.ds(base, per_w)], idx_v)

        def gather_copy(g, kb):
            return pltpu.make_async_copy(
                table_hbm.at[idx_v.at[g]], rows_v.at[kb], sg[kb])

        def off_copy(g, kb):
            return pltpu.make_async_copy(
                off_hbm.at[base + g], off_v.at[kb], sf[kb])

        def out_copy(g, ob):
            gi = base + g
            l = gi // cpl
            bt = lax.rem(gi, cpl)
            return pltpu.make_async_copy(
                q_v.at[ob, :, :, pl.ds(0, _G)],
                out_hbm.at[l, :, bt], so[ob])

        iota16 = lax.iota(jnp.int32, 16)
        d_rows = [iota16 + (dc * _LANES) for dc in range(dc_n)]
        d_hi = [r // 8 for r in d_rows]
        d_lo = [lax.rem(r, 8) for r in d_rows]
        zeros16 = jnp.zeros((16,), jnp.int32)

        def compute(g, kb, ob):
            gi = base + g
            kb_splat = zeros16 + kb
            l_splat = zeros16 + gi // cpl
            rows = rows_v.at[kb]
            q = q_v.at[ob]
            pe_regs = [
                plsc.load_gather(pe_v, [l_splat, d_rows[dc]])
                for dc in range(dc_n)
            ]

            @plsc.parallel_loop(0, _G, unroll=8)
            def _(b):
                b_splat = zeros16 + b
                h_splat = plsc.load_gather(off_v, [kb_splat, b_splat])
                for dc in range(dc_n):
                    vals = plsc.load_gather(
                        rows, [b_splat, d_rows[dc] + h_splat])
                    plsc.store_scatter(
                        q, [d_hi[dc], d_lo[dc], b_splat],
                        vals + pe_regs[dc])

        for j in range(_NBUF - 1):
            gather_copy(j, j).start()
            off_copy(j, j).start()

        @pl.loop(0, per_w, step=_NBUF)
        def _(g0):
            for kk in range(_NBUF):
                g = g0 + kk
                gather_copy(g, kk).wait()
                off_copy(g, kk).wait()

                @pl.when(g + (_NBUF - 1) < per_w)
                def _():
                    gather_copy(g + (_NBUF - 1), (kk + (_NBUF - 1)) % _NBUF).start()
                    off_copy(g + (_NBUF - 1), (kk + (_NBUF - 1)) % _NBUF).start()

                @pl.when(g >= 2)
                def _():
                    out_copy(g - 2, kk % 2).wait()

                compute(g, kb=kk, ob=kk % 2)
                out_copy(g, kk % 2).start()

        out_copy(per_w - 2, (per_w - 2) % 2).wait()
        out_copy(per_w - 1, (per_w - 1) % 2).wait()

    return k(table, idx2, off2, pe2)


_TBLK = 8192


def _tc_pack_body(t_ref, out_ref):
    xT = t_ref[...].T
    half = xT.shape[0] // 2
    out_ref[:, : xT.shape[1]] = xT[:half]
    out_ref[:, xT.shape[1]:] = xT[half:]


def _tc_pack(tT):
    emb, vocab = tT.shape
    nblk = pl.cdiv(vocab, _TBLK)
    return pl.pallas_call(
        _tc_pack_body,
        grid=(nblk,),
        in_specs=[pl.BlockSpec((emb, _TBLK), lambda i: (0, i))],
        out_specs=pl.BlockSpec((_TBLK // 2, 2 * emb), lambda i: (i, 0)),
        out_shape=jax.ShapeDtypeStruct((nblk * (_TBLK // 2), 2 * emb),
                                       jnp.float32),
    )(tT)


def kernel(input, table, pe):
    batch, seq = input.shape
    emb = table.shape[1]
    idx2 = input.T.reshape(seq * batch // _G, _G)
    pe2 = pe.reshape(pe.shape[0], emb)[:seq]
    half = _TBLK // 2
    j = idx2 % _TBLK
    off2 = (j // half) * emb
    idx2 = (idx2 // _TBLK) * half + (j % half)
    tpack = _tc_pack(table.T)
    q = _sc_embed(tpack, idx2, off2, pe2, seq=seq, batch=batch, emb=emb)
    return q.transpose(0, 2, 4, 1, 3).reshape(seq, batch, emb)


# speedup vs baseline: 1.8042x; 1.0604x over previous
import dataclasses
import functools

import jax
import jax.numpy as jnp
from jax import lax
from jax.experimental import pallas as pl
from jax.experimental.pallas import tpu as pltpu
from jax.experimental.pallas import tpu_sc as plsc

_NC = 2
_NS = 16
_NW = _NC * _NS
_G = 128
_LANES = 16
_NBUF = 4
_QS = 129


def _compiler_params():
    cp = pltpu.CompilerParams(use_tc_tiling_on_sc=False)
    if "needs_layout_passes" in pltpu.CompilerParams.__dataclass_fields__:
        cp = dataclasses.replace(cp, needs_layout_passes=False)
    return cp


@functools.partial(jax.jit, static_argnames=("seq", "batch", "emb"))
def _sc_embed(table, idx2, off2, pe2, *, seq, batch, emb):
    n_flat = seq * batch
    nchunks = n_flat // _G
    per_w = nchunks // _NW
    cpl = batch // _G
    dt_n = emb // 8
    dc_n = emb // _LANES
    emb_pad = table.shape[1]

    mesh = plsc.VectorSubcoreMesh(core_axis_name="c", subcore_axis_name="s")

    @functools.partial(
        pl.kernel,
        out_type=jax.ShapeDtypeStruct((seq, dt_n, cpl, 8, _G), jnp.float32),
        mesh=mesh,
        scratch_types=[
            pltpu.VMEM((seq, emb), jnp.float32),
            pltpu.VMEM((per_w, _G), jnp.int32),
            pltpu.VMEM((_NBUF, _G), jnp.int32),
            pltpu.VMEM((_NBUF, _G, emb_pad), jnp.float32),
            pltpu.VMEM((2, emb // 8, 8, _QS), jnp.float32),
            pltpu.SemaphoreType.DMA,
            pltpu.SemaphoreType.DMA,
            pltpu.SemaphoreType.DMA,
            pltpu.SemaphoreType.DMA,
            pltpu.SemaphoreType.DMA,
            pltpu.SemaphoreType.DMA,
            pltpu.SemaphoreType.DMA,
            pltpu.SemaphoreType.DMA,
            pltpu.SemaphoreType.DMA,
            pltpu.SemaphoreType.DMA,
        ],
        compiler_params=_compiler_params(),
    )
    def k(table_hbm, idx_hbm, off_hbm, pe_hbm, out_hbm, pe_v, idx_v, off_v,
          rows_v, q_v, sg0, sg1, sg2, sg3, so0, so1, sf0, sf1, sf2, sf3):
        sg = (sg0, sg1, sg2, sg3)
        so = (so0, so1)
        sf = (sf0, sf1, sf2, sf3)
        wid = lax.axis_index("s") * _NC + lax.axis_index("c")
        base = wid * per_w
        pltpu.sync_copy(pe_hbm, pe_v)
        pltpu.sync_copy(idx_hbm.at[pl.ds(base, per_w)], idx_v)

        def gather_copy(g, kb):
            return pltpu.make_async_copy(
                table_hbm.at[idx_v.at[g]], rows_v.at[kb], sg[kb])

        def off_copy(g, kb):
            return pltpu.make_async_copy(
                off_hbm.at[base + g], off_v.at[kb], sf[kb])

        def out_copy(g, ob):
            gi = base + g
            l = gi // cpl
            bt = lax.rem(gi, cpl)
            return pltpu.make_async_copy(
                q_v.at[ob, :, :, pl.ds(0, _G)],
                out_hbm.at[l, :, bt], so[ob])

        iota16 = lax.iota(jnp.int32, 16)
        d_rows = [iota16 + (dc * _LANES) for dc in range(dc_n)]
        d_hi = [r // 8 for r in d_rows]
        d_lo = [lax.rem(r, 8) for r in d_rows]
        zeros16 = jnp.zeros((16,), jnp.int32)

        def compute(g, kb, ob):
            gi = base + g
            kb_splat = zeros16 + kb
            l_splat = zeros16 + gi // cpl
            rows = rows_v.at[kb]
            q = q_v.at[ob]
            pe_regs = [
                plsc.load_gather(pe_v, [l_splat, d_rows[dc]])
                for dc in range(dc_n)
            ]

            @plsc.parallel_loop(0, _G, unroll=8)
            def _(b):
                b_splat = zeros16 + b
                h_splat = plsc.load_gather(off_v, [kb_splat, b_splat])
                for dc in range(dc_n):
                    vals = plsc.load_gather(
                        rows, [b_splat, d_rows[dc] + h_splat])
                    plsc.store_scatter(
                        q, [d_hi[dc], d_lo[dc], b_splat],
                        vals + pe_regs[dc])

        for j in range(_NBUF - 1):
            gather_copy(j, j).start()
            off_copy(j, j).start()

        @pl.loop(0, per_w, step=_NBUF)
        def _(g0):
            for kk in range(_NBUF):
                g = g0 + kk
                gather_copy(g, kk).wait()
                off_copy(g, kk).wait()

                @pl.when(g + (_NBUF - 1) < per_w)
                def _():
                    gather_copy(g + (_NBUF - 1), (kk + (_NBUF - 1)) % _NBUF).start()
                    off_copy(g + (_NBUF - 1), (kk + (_NBUF - 1)) % _NBUF).start()

                @pl.when(g >= 2)
                def _():
                    out_copy(g - 2, kk % 2).wait()

                compute(g, kb=kk, ob=kk % 2)
                out_copy(g, kk % 2).start()

        out_copy(per_w - 2, (per_w - 2) % 2).wait()
        out_copy(per_w - 1, (per_w - 1) % 2).wait()

    return k(table, idx2, off2, pe2)


_TBLK = 16384


def _tc_pack_body(t_ref, out_ref):
    xT = t_ref[...].T
    half = xT.shape[0] // 2
    out_ref[:, : xT.shape[1]] = xT[:half]
    out_ref[:, xT.shape[1]:] = xT[half:]


def _tc_pack(tT):
    emb, vocab = tT.shape
    nblk = pl.cdiv(vocab, _TBLK)
    return pl.pallas_call(
        _tc_pack_body,
        grid=(nblk,),
        in_specs=[pl.BlockSpec((emb, _TBLK), lambda i: (0, i))],
        out_specs=pl.BlockSpec((_TBLK // 2, 2 * emb), lambda i: (i, 0)),
        out_shape=jax.ShapeDtypeStruct((nblk * (_TBLK // 2), 2 * emb),
                                       jnp.float32),
    )(tT)


def kernel(input, table, pe):
    batch, seq = input.shape
    emb = table.shape[1]
    idx2 = input.T.reshape(seq * batch // _G, _G)
    pe2 = pe.reshape(pe.shape[0], emb)[:seq]
    half = _TBLK // 2
    j = idx2 % _TBLK
    off2 = (j // half) * emb
    idx2 = (idx2 // _TBLK) * half + (j % half)
    tpack = _tc_pack(table.T)
    q = _sc_embed(tpack, idx2, off2, pe2, seq=seq, batch=batch, emb=emb)
    return q.transpose(0, 2, 4, 1, 3).reshape(seq, batch, emb)


# speedup vs baseline: 1.8569x; 1.0292x over previous
import dataclasses
import functools

import jax
import jax.numpy as jnp
from jax import lax
from jax.experimental import pallas as pl
from jax.experimental.pallas import tpu as pltpu
from jax.experimental.pallas import tpu_sc as plsc

_NC = 2
_NS = 16
_NW = _NC * _NS
_G = 128
_LANES = 16
_NBUF = 4
_QS = 129


def _compiler_params():
    cp = pltpu.CompilerParams(use_tc_tiling_on_sc=False)
    if "needs_layout_passes" in pltpu.CompilerParams.__dataclass_fields__:
        cp = dataclasses.replace(cp, needs_layout_passes=False)
    return cp


@functools.partial(jax.jit, static_argnames=("seq", "batch", "emb"))
def _sc_embed(table, idx2, off2, pe2, *, seq, batch, emb):
    n_flat = seq * batch
    nchunks = n_flat // _G
    per_w = nchunks // _NW
    cpl = batch // _G
    dt_n = emb // 8
    dc_n = emb // _LANES
    emb_pad = table.shape[1]

    mesh = plsc.VectorSubcoreMesh(core_axis_name="c", subcore_axis_name="s")

    @functools.partial(
        pl.kernel,
        out_type=jax.ShapeDtypeStruct((seq, dt_n, cpl, 8, _G), jnp.float32),
        mesh=mesh,
        scratch_types=[
            pltpu.VMEM((seq, emb), jnp.float32),
            pltpu.VMEM((per_w, _G), jnp.int32),
            pltpu.VMEM((_NBUF, _G), jnp.int32),
            pltpu.VMEM((_NBUF, _G, emb_pad), jnp.float32),
            pltpu.VMEM((2, emb // 8, 8, _QS), jnp.float32),
            pltpu.SemaphoreType.DMA,
            pltpu.SemaphoreType.DMA,
            pltpu.SemaphoreType.DMA,
            pltpu.SemaphoreType.DMA,
            pltpu.SemaphoreType.DMA,
            pltpu.SemaphoreType.DMA,
            pltpu.SemaphoreType.DMA,
            pltpu.SemaphoreType.DMA,
            pltpu.SemaphoreType.DMA,
            pltpu.SemaphoreType.DMA,
        ],
        compiler_params=_compiler_params(),
    )
    def k(table_hbm, idx_hbm, off_hbm, pe_hbm, out_hbm, pe_v, idx_v, off_v,
          rows_v, q_v, sg0, sg1, sg2, sg3, so0, so1, sf0, sf1, sf2, sf3):
        sg = (sg0, sg1, sg2, sg3)
        so = (so0, so1)
        sf = (sf0, sf1, sf2, sf3)
        wid = lax.axis_index("s") * _NC + lax.axis_index("c")
        base = wid * per_w
        pltpu.sync_copy(pe_hbm, pe_v)
        pltpu.sync_copy(idx_hbm.at[pl.ds(base, per_w)], idx_v)

        def gather_copy(g, kb):
            return pltpu.make_async_copy(
                table_hbm.at[idx_v.at[g]], rows_v.at[kb], sg[kb])

        def off_copy(g, kb):
            return pltpu.make_async_copy(
                off_hbm.at[base + g], off_v.at[kb], sf[kb])

        def out_copy(g, ob):
            gi = base + g
            l = gi // cpl
            bt = lax.rem(gi, cpl)
            return pltpu.make_async_copy(
                q_v.at[ob, :, :, pl.ds(0, _G)],
                out_hbm.at[l, :, bt], so[ob])

        iota16 = lax.iota(jnp.int32, 16)
        d_rows = [iota16 + (dc * _LANES) for dc in range(dc_n)]
        d_hi = [r // 8 for r in d_rows]
        d_lo = [lax.rem(r, 8) for r in d_rows]
        zeros16 = jnp.zeros((16,), jnp.int32)

        def compute(g, kb, ob):
            gi = base + g
            kb_splat = zeros16 + kb
            l_splat = zeros16 + gi // cpl
            rows = rows_v.at[kb]
            q = q_v.at[ob]
            pe_regs = [
                plsc.load_gather(pe_v, [l_splat, d_rows[dc]])
                for dc in range(dc_n)
            ]

            @plsc.parallel_loop(0, _G, unroll=8)
            def _(b):
                b_splat = zeros16 + b
                h_splat = plsc.load_gather(off_v, [kb_splat, b_splat])
                for dc in range(dc_n):
                    vals = plsc.load_gather(
                        rows, [b_splat, d_rows[dc] + h_splat])
                    plsc.store_scatter(
                        q, [d_hi[dc], d_lo[dc], b_splat],
                        vals + pe_regs[dc])

        for j in range(_NBUF - 1):
            gather_copy(j, j).start()
            off_copy(j, j).start()

        @pl.loop(0, per_w, step=_NBUF)
        def _(g0):
            for kk in range(_NBUF):
                g = g0 + kk
                gather_copy(g, kk).wait()
                off_copy(g, kk).wait()

                @pl.when(g + (_NBUF - 1) < per_w)
                def _():
                    gather_copy(g + (_NBUF - 1), (kk + (_NBUF - 1)) % _NBUF).start()
                    off_copy(g + (_NBUF - 1), (kk + (_NBUF - 1)) % _NBUF).start()

                @pl.when(g >= 2)
                def _():
                    out_copy(g - 2, kk % 2).wait()

                compute(g, kb=kk, ob=kk % 2)
                out_copy(g, kk % 2).start()

        out_copy(per_w - 2, (per_w - 2) % 2).wait()
        out_copy(per_w - 1, (per_w - 1) % 2).wait()

    return k(table, idx2, off2, pe2)


_TBLK = 32768


def _tc_pack_body(t_ref, out_ref):
    xT = t_ref[...].T
    half = xT.shape[0] // 2
    out_ref[:, : xT.shape[1]] = xT[:half]
    out_ref[:, xT.shape[1]:] = xT[half:]


def _tc_pack(tT):
    emb, vocab = tT.shape
    nblk = pl.cdiv(vocab, _TBLK)
    return pl.pallas_call(
        _tc_pack_body,
        grid=(nblk,),
        in_specs=[pl.BlockSpec((emb, _TBLK), lambda i: (0, i))],
        out_specs=pl.BlockSpec((_TBLK // 2, 2 * emb), lambda i: (i, 0)),
        out_shape=jax.ShapeDtypeStruct((nblk * (_TBLK // 2), 2 * emb),
                                       jnp.float32),
    )(tT)


def kernel(input, table, pe):
    batch, seq = input.shape
    emb = table.shape[1]
    idx2 = input.T.reshape(seq * batch // _G, _G)
    pe2 = pe.reshape(pe.shape[0], emb)[:seq]
    half = _TBLK // 2
    j = idx2 % _TBLK
    off2 = (j // half) * emb
    idx2 = (idx2 // _TBLK) * half + (j % half)
    tpack = _tc_pack(table.T)
    q = _sc_embed(tpack, idx2, off2, pe2, seq=seq, batch=batch, emb=emb)
    return q.transpose(0, 2, 4, 1, 3).reshape(seq, batch, emb)
